# baseline (speedup 1.0000x reference)
import jax
import jax.numpy as jnp
from jax.experimental import pallas as pl
from jax.experimental.pallas import tpu as pltpu


def _dot(a, b, dims):
    return jax.lax.dot_general(a, b, (dims, ((), ())),
                               preferred_element_type=jnp.float32)


def _gnn_kernel(x_ref, mask_ref, w1_ref, b1_ref, wl1_ref, bl1_ref, wr1_ref,
                wl2_ref, bl2_ref, wr2_ref, out_ref):
    i = pl.program_id(0)
    n, hdim = x_ref.shape[1], x_ref.shape[2]

    h = (_dot(x_ref[0], w1_ref[...], (((1,), (1,))))
         + b1_ref[...].reshape(1, b1_ref.shape[0]))

    inv_cnt = 1.0 / (n - 1)
    nrm2 = jnp.sum(h * h, axis=1, keepdims=True)
    rn = 1.0 / jnp.maximum(jnp.sqrt(nrm2), 1e-8)
    u = h * rn
    c = nrm2 * (rn * rn)

    s1 = _dot(u, _dot(u, h, (((0,), (0,)))), (((1,), (0,))))
    agg1 = (s1 - c * h) * inv_cnt
    o1 = jnp.maximum(
        _dot(agg1, wl1_ref[...], (((1,), (1,))))
        + _dot(h, wr1_ref[...], (((1,), (1,))))
        + bl1_ref[...].reshape(1, bl1_ref.shape[0]), 0.0)

    s2 = _dot(u, _dot(u, o1, (((0,), (0,)))), (((1,), (0,))))
    agg2 = (s2 - c * o1) * inv_cnt
    z = (_dot(wl2_ref[...], agg2, (((1,), (1,))))
         + _dot(wr2_ref[...], o1, (((1,), (1,))))
         + bl2_ref[...].reshape(1, 1))
    out_ref[pl.ds(i, 1), :] = jax.nn.sigmoid(z) * mask_ref[pl.ds(i, 1), :]


@jax.jit
def kernel(x, mask_cls, W1, b1, Wl1, bl1, Wr1, Wl2, bl2, Wr2):
    B, N, H = x.shape
    whole = lambda a: pl.BlockSpec(a.shape, lambda i: (0,) * a.ndim)
    return pl.pallas_call(
        _gnn_kernel,
        grid=(B,),
        in_specs=[
            pl.BlockSpec((1, N, H), lambda i: (i, 0, 0)),
            whole(mask_cls), whole(W1), whole(b1), whole(Wl1), whole(bl1),
            whole(Wr1), whole(Wl2), whole(bl2), whole(Wr2),
        ],
        out_specs=pl.BlockSpec((B, N), lambda i: (0, 0)),
        out_shape=jax.ShapeDtypeStruct((B, N), jnp.float32),
        compiler_params=pltpu.CompilerParams(
            dimension_semantics=("arbitrary",)),
    )(x, mask_cls, W1, b1, Wl1, bl1, Wr1, Wl2, bl2, Wr2)
